# baseline (speedup 1.0000x reference)
import jax
import jax.numpy as jnp
from jax import lax
from jax.experimental import pallas as pl
from jax.experimental.pallas import tpu as pltpu
from jax.experimental.pallas import tpu_sc as plsc

_N_USERS = 100000
_N_ITEMS = 1000000
_DIM = 32
_NW = 32
_CR = 496
_SPLIT = 500000


def _copy_table(src, dst, wid, bufs, sems_in, sems_out, total_rows):
    nch = total_rows // _CR
    nfull = nch // _NW
    nextra = nch - nfull * _NW
    rem_rows = total_rows - nch * _CR
    rem_base = nch * _CR

    def chunk(ref, j):
        return ref.at[pl.ds(j * _CR, _CR), :]

    def start_in(t, b):
        return pltpu.make_async_copy(
            chunk(src, t * _NW + wid), bufs.at[b], sems_in.at[b]
        )

    def start_out(t, b):
        return pltpu.make_async_copy(
            bufs.at[b], chunk(dst, t * _NW + wid), sems_out.at[b]
        )

    h_in = [None, None]
    h_out = [None, None]
    if nfull > 0:
        h_in[0] = start_in(0, 0)
        h_in[0].start()
    for t in range(nfull):
        cur, nxt = t % 2, (t + 1) % 2
        if t + 1 < nfull:
            if h_out[nxt] is not None:
                h_out[nxt].wait()
            h_in[nxt] = start_in(t + 1, nxt)
            h_in[nxt].start()
        h_in[cur].wait()
        h_out[cur] = start_out(t, cur)
        h_out[cur].start()
    for h in h_out:
        if h is not None:
            h.wait()
    if nextra:
        @pl.when(wid < nextra)
        def _tail():
            j = nfull * _NW + wid
            pltpu.sync_copy(chunk(src, j), bufs.at[0])
            pltpu.sync_copy(bufs.at[0], chunk(dst, j))
    if rem_rows:
        @pl.when(wid == _NW - 1)
        def _remainder():
            pltpu.sync_copy(
                src.at[pl.ds(rem_base, rem_rows), :],
                bufs.at[1, pl.ds(0, rem_rows), :],
            )
            pltpu.sync_copy(
                bufs.at[1, pl.ds(0, rem_rows), :],
                dst.at[pl.ds(rem_base, rem_rows), :],
            )


def _sc_body(u_in, i_in, u_out, i_half_out, bufs, sem_in, sem_out):
    wid = lax.axis_index("s") * 2 + lax.axis_index("c")
    _copy_table(i_in, i_half_out, wid, bufs, sem_in, sem_out, _SPLIT)
    _copy_table(u_in, u_out, wid, bufs, sem_in, sem_out, _N_USERS)


def _sc_copy(user_weight, item_weight):
    mesh = plsc.VectorSubcoreMesh(core_axis_name="c", subcore_axis_name="s")
    run = pl.kernel(
        _sc_body,
        out_type=(
            jax.ShapeDtypeStruct((_N_USERS, _DIM), jnp.float32),
            jax.ShapeDtypeStruct((_SPLIT, _DIM), jnp.float32),
        ),
        mesh=mesh,
        scratch_types=[
            pltpu.VMEM((2, _CR, _DIM), jnp.float32),
            pltpu.SemaphoreType.DMA((2,)),
            pltpu.SemaphoreType.DMA((2,)),
        ],
    )
    return run(user_weight, item_weight)


def _tc_block(src_ref, dst_ref):
    dst_ref[...] = src_ref[...]


def _tc_copy_upper(item_weight):
    rows = _N_ITEMS - _SPLIT
    br = 10000
    off = _SPLIT // br
    return pl.pallas_call(
        _tc_block,
        out_shape=jax.ShapeDtypeStruct((rows, _DIM), jnp.float32),
        grid=(rows // br,),
        in_specs=[pl.BlockSpec((br, _DIM), lambda i: (i + off, 0))],
        out_specs=pl.BlockSpec((br, _DIM), lambda i: (i, 0)),
    )(item_weight)


def kernel(adj, user_weight, item_weight):
    del adj
    u_out, i_lower = _sc_copy(user_weight, item_weight)
    i_upper = _tc_copy_upper(item_weight)
    return (u_out, jnp.concatenate([i_lower, i_upper], axis=0))


# speedup vs baseline: 1.0817x; 1.0817x over previous
import jax
import jax.numpy as jnp
from jax import lax
from jax.experimental import pallas as pl
from jax.experimental.pallas import tpu as pltpu
from jax.experimental.pallas import tpu_sc as plsc

_N_USERS = 100000
_N_ITEMS = 1000000
_DIM = 32
_NW = 32
_CR = 1000


def _copy_table(src, dst, wid, buf, total_rows):
    nch = total_rows // _CR
    nfull = nch // _NW
    nextra = nch - nfull * _NW

    def chunk(ref, j):
        return ref.at[pl.ds(j * _CR, _CR), :]

    for t in range(nfull):
        pltpu.sync_copy(chunk(src, t * _NW + wid), buf)
        pltpu.sync_copy(buf, chunk(dst, t * _NW + wid))
    if nextra:
        @pl.when(wid < nextra)
        def _tail():
            j = nfull * _NW + wid
            pltpu.sync_copy(chunk(src, j), buf)
            pltpu.sync_copy(buf, chunk(dst, j))


def _copy_body(u_in, i_in, u_out, i_out, shared):
    sid = lax.axis_index("s")
    wid = sid * 2 + lax.axis_index("c")
    buf = shared.at[sid]
    _copy_table(i_in, i_out, wid, buf, _N_ITEMS)
    _copy_table(u_in, u_out, wid, buf, _N_USERS)


@jax.jit
def _sc_copy(user_weight, item_weight):
    mesh = plsc.VectorSubcoreMesh(core_axis_name="c", subcore_axis_name="s")
    run = pl.kernel(
        _copy_body,
        out_type=(
            jax.ShapeDtypeStruct((_N_USERS, _DIM), jnp.float32),
            jax.ShapeDtypeStruct((_N_ITEMS, _DIM), jnp.float32),
        ),
        mesh=mesh,
        scratch_types=[
            pltpu.VMEM_SHARED((16, _CR, _DIM), jnp.float32),
        ],
    )
    return run(user_weight, item_weight)


def kernel(adj, user_weight, item_weight):
    del adj
    return _sc_copy(user_weight, item_weight)
